# Initial kernel scaffold; baseline (speedup 1.0000x reference)
import jax
import jax.numpy as jnp
from jax.experimental import pallas as pl


def kernel(field_indices, edge_index, batch_idx, emb_tables, aligner_W, aligner_b, mask_logits, gat1_W, gat1_att_src, gat1_att_dst, gat1_b, gat2_W, gat2_att_src, gat2_att_dst, gat2_b, cls_W1, cls_b1, cls_W2, cls_b2):
    raise NotImplementedError("write your pallas kernel here")


# speedup vs baseline: 1.7861x; 1.7861x over previous
import functools

import jax
import jax.numpy as jnp
import numpy as np
from jax.experimental import pallas as pl
from jax.experimental.pallas import tpu as pltpu

F = 16
G = 3125
VOCAB = 1000
E_DIM = 64
H = 128
HEADS = 4
N = F * G
E = 93750


def _mm_kernel(x_ref, w_ref, o_ref):
    o_ref[...] = jnp.dot(x_ref[...], w_ref[...], preferred_element_type=jnp.float32)


def _mm(x, w, block_rows=2000):
    m, k = x.shape
    _, n = w.shape
    grid = (m // block_rows,)
    return pl.pallas_call(
        _mm_kernel,
        grid=grid,
        in_specs=[
            pl.BlockSpec((block_rows, k), lambda i: (i, 0)),
            pl.BlockSpec((k, n), lambda i: (0, 0)),
        ],
        out_specs=pl.BlockSpec((block_rows, n), lambda i: (i, 0)),
        out_shape=jax.ShapeDtypeStruct((m, n), jnp.float32),
    )(x, w)


def _cls_kernel(p_ref, w1_ref, b1_ref, w2_ref, b2_ref, o_ref):
    h1 = jnp.dot(p_ref[...], w1_ref[...], preferred_element_type=jnp.float32)
    h1 = h1 + b1_ref[...]
    h1 = jnp.where(h1 > 0, h1, 0.01 * h1)
    o_ref[...] = jnp.dot(h1, w2_ref[...], preferred_element_type=jnp.float32) + b2_ref[...]


def kernel(field_indices, edge_index, batch_idx, emb_tables, aligner_W, aligner_b, mask_logits, gat1_W, gat1_att_src, gat1_att_dst, gat1_b, gat2_W, gat2_att_src, gat2_att_dst, gat2_b, cls_W1, cls_b1, cls_W2, cls_b2):
    flat_tab = emb_tables.reshape(F * VOCAB, E_DIM)
    idx = field_indices.T + (jnp.arange(F) * VOCAB)[None, :]
    emb = jnp.take(flat_tab, idx.reshape(-1), axis=0)
    gate = jax.nn.sigmoid(mask_logits)
    x = _mm(emb, aligner_W) + aligner_b
    x = x * jnp.tile(gate, G)[:, None]

    src = edge_index[0]
    dst = edge_index[1]

    h = _mm(x, gat1_W)
    h4 = h.reshape(N, HEADS, H)
    a_src = jnp.einsum("nhc,hc->nh", h4, gat1_att_src)
    a_dst = jnp.einsum("nhc,hc->nh", h4, gat1_att_dst)
    m1 = jnp.max(a_src, axis=0) + jnp.max(a_dst, axis=0)
    alpha_e = jax.nn.leaky_relu(a_src[src] + a_dst[dst], 0.2)
    ex_e = jnp.exp(alpha_e - m1[None, :])
    ex_loop = jnp.exp(jax.nn.leaky_relu(a_src + a_dst, 0.2) - m1[None, :])
    denom = jax.ops.segment_sum(ex_e, dst, num_segments=N) + ex_loop
    num = jax.ops.segment_sum(h4[src] * ex_e[:, :, None], dst, num_segments=N)
    num = num + h4 * ex_loop[:, :, None]
    out1 = num / (denom + 1e-16)[:, :, None]
    x1 = out1.reshape(N, HEADS * H) + gat1_b
    x1 = jnp.where(x1 > 0, x1, jnp.expm1(x1))

    h2 = _mm(x1, gat2_W)
    a_src2 = h2 @ gat2_att_src[0]
    a_dst2 = h2 @ gat2_att_dst[0]
    m2 = jnp.max(a_src2) + jnp.max(a_dst2)
    alpha2 = jax.nn.leaky_relu(a_src2[src] + a_dst2[dst], 0.2)
    ex2 = jnp.exp(alpha2 - m2)
    exl2 = jnp.exp(jax.nn.leaky_relu(a_src2 + a_dst2, 0.2) - m2)
    denom2 = jax.ops.segment_sum(ex2, dst, num_segments=N) + exl2
    num2 = jax.ops.segment_sum(h2[src] * ex2[:, None], dst, num_segments=N)
    num2 = num2 + h2 * exl2[:, None]
    x2 = num2 / (denom2 + 1e-16)[:, None] + gat2_b

    counts = jax.ops.segment_sum(jnp.ones((N,), jnp.float32), batch_idx, num_segments=G)
    pooled = jax.ops.segment_sum(x2, batch_idx, num_segments=G)
    pooled = pooled / jnp.clip(counts, 1.0, None)[:, None]

    pooled_pad = jnp.pad(pooled, ((0, 3), (0, 0)))
    logits = pl.pallas_call(
        _cls_kernel,
        in_specs=[
            pl.BlockSpec((3128, H), lambda: (0, 0)),
            pl.BlockSpec((H, H // 2), lambda: (0, 0)),
            pl.BlockSpec((H // 2,), lambda: (0,)),
            pl.BlockSpec((H // 2, 8), lambda: (0, 0)),
            pl.BlockSpec((8,), lambda: (0,)),
        ],
        out_specs=pl.BlockSpec((3128, 8), lambda: (0, 0)),
        out_shape=jax.ShapeDtypeStruct((3128, 8), jnp.float32),
    )(pooled_pad, cls_W1, cls_b1, cls_W2, cls_b2)[:G]
    return (logits, gate)
